# baseline (speedup 1.0000x reference)
import math

import jax
import jax.numpy as jnp
from jax.experimental import pallas as pl
from jax.experimental.pallas import tpu as pltpu

B, N, H, W = 16, 20, 224, 224
HID = 1024
HEADS = 4
HD = HID // HEADS
SCALE = float(math.sqrt(float(HD)))
INV_HW = 1.0 / float(H * W)
BF = jnp.bfloat16
F32 = jnp.float32


def _dot(a, b):
    return jnp.dot(a.astype(BF) if a.dtype != BF else a,
                   b.astype(BF) if b.dtype != BF else b,
                   preferred_element_type=F32)


def _dot_t(a, b):
    return jax.lax.dot_general(a, b, (((1,), (1,)), ((), ())),
                               preferred_element_type=F32)


def _mask_body(o, t, bb, objsum_ref, overlap_ref, iou_ref, i):

    objsum = jnp.sum(o, axis=(1, 2))
    overlap = jnp.sum(o * t[None, :, :], axis=(1, 2))

    x1 = jnp.floor(bb[0])
    y1 = jnp.floor(bb[1])
    x2 = jnp.floor(bb[2])
    y2 = jnp.floor(bb[3])

    hi = jax.lax.broadcasted_iota(jnp.int32, (H, N), 0).astype(F32)
    rowm = jnp.where((hi >= y1[None, :]) & (hi < y2[None, :]), 1.0, 0.0)
    wi = jax.lax.broadcasted_iota(jnp.int32, (W, N), 0).astype(F32)
    colm = jnp.where((wi >= x1[None, :]) & (wi < x2[None, :]), 1.0, 0.0)

    tmp = jnp.dot(t, colm, preferred_element_type=F32)
    inter = jnp.sum(rowm * tmp, axis=0)
    box_area = jnp.sum(rowm, axis=0) * jnp.sum(colm, axis=0)
    t_area = jnp.sum(t)
    iou = inter / (box_area + t_area - inter + 1e-8)

    objsum_ref[i, 0] = objsum
    overlap_ref[i, 0] = overlap
    iou_ref[i, 0] = iou


def _mask_kernel(obj_ref, tgt_ref, bb_ref, objsum_ref, overlap_ref, iou_ref):
    for i in range(4):
        _mask_body(obj_ref[i], tgt_ref[i], bb_ref[i],
                   objsum_ref, overlap_ref, iou_ref, i)


def _mask_pass(obj, tgt, bb_t):
    out_sds = jax.ShapeDtypeStruct((B, 1, N), F32)
    return pl.pallas_call(
        _mask_kernel,
        grid=(B // 4,),
        in_specs=[
            pl.BlockSpec((4, N, H, W), lambda b: (b, 0, 0, 0)),
            pl.BlockSpec((4, H, W), lambda b: (b, 0, 0)),
            pl.BlockSpec((4, 4, N), lambda b: (b, 0, 0)),
        ],
        out_specs=[
            pl.BlockSpec((4, 1, N), lambda b: (b, 0, 0)),
            pl.BlockSpec((4, 1, N), lambda b: (b, 0, 0)),
            pl.BlockSpec((4, 1, N), lambda b: (b, 0, 0)),
        ],
        out_shape=[out_sds, out_sds, out_sds],
        compiler_params=pltpu.CompilerParams(
            dimension_semantics=("parallel",),
            vmem_limit_bytes=52 * 1024 * 1024,
        ),
        name="mask_pass",
    )(obj, tgt, bb_t)


def _front_kernel(objc_ref, bbw1_ref, bbb1_ref, bbfw_ref, bbfb_ref,
                  ov_ref, iou_ref, w1ov_ref, w1iou_ref, orb1_ref,
                  org_ref, orbe_ref, orm_ref, orv_ref,
                  qw_ref, qb_ref,
                  qfeat_ref, r_ref):
    f = (objc_ref[...] * INV_HW).astype(BF).astype(F32)
    w1b = bbw1_ref[...].astype(BF).astype(F32)
    w1s = jnp.sum(w1b, axis=0, keepdims=True)
    h = jax.nn.relu(f * w1s + bbb1_ref[...])
    of = _dot(h, bbfw_ref[...]) + bbfb_ref[...]
    q = _dot(of, qw_ref[...]) + qb_ref[...]
    qfeat_ref[...] = q.astype(BF)

    e = (_dot(ov_ref[...], w1ov_ref[...])
         + _dot(iou_ref[...], w1iou_ref[...])
         + orb1_ref[...])
    bn = (e - orm_ref[...]) * jax.lax.rsqrt(orv_ref[...] + 1e-5) * org_ref[...] + orbe_ref[...]
    r_ref[...] = jax.nn.relu(bn).astype(BF)


def _front_pass(objsum_col, bb_w1, bb_b1, bb_fw, bb_fb,
                overlap, iou, w1_ov, w1_iou, or_b1, or_g, or_be, or_m, or_v,
                q_w, q_b):
    return pl.pallas_call(
        _front_kernel,
        out_shape=[
            jax.ShapeDtypeStruct((N * B, HID), BF),
            jax.ShapeDtypeStruct((B, HID), BF),
        ],
        name="front_pass",
    )(objsum_col, bb_w1, bb_b1, bb_fw, bb_fb,
      overlap, iou, w1_ov, w1_iou, or_b1, or_g, or_be, or_m, or_v, q_w, q_b)


def _kv_kernel(r_ref, w2_ref, b2_ref, kw_ref, kb_ref, vw_ref, vb_ref,
               k_ref, v_ref):
    rb = r_ref[...].astype(BF)
    kwb = kw_ref[...].astype(BF)
    vwb = vw_ref[...].astype(BF)
    for i in range(4):
        rel = _dot(rb, w2_ref[:, i * HID:(i + 1) * HID]) + b2_ref[i]
        k_ref[i] = (_dot(rel, kwb) + kb_ref[...]).astype(BF)
        v_ref[i] = (_dot(rel, vwb) + vb_ref[...]).astype(BF)


def _kv_pass(r, or_w2, or_b2_3d, k_w, k_b, v_w, v_b):
    out_sds = jax.ShapeDtypeStruct((N, B, HID), BF)
    return pl.pallas_call(
        _kv_kernel,
        grid=(N // 4,),
        in_specs=[
            pl.BlockSpec((B, HID), lambda n: (0, 0)),
            pl.BlockSpec((HID, 4 * HID), lambda n: (0, n)),
            pl.BlockSpec((4, 1, HID), lambda n: (n, 0, 0)),
            pl.BlockSpec((HID, HID), lambda n: (0, 0)),
            pl.BlockSpec((1, HID), lambda n: (0, 0)),
            pl.BlockSpec((HID, HID), lambda n: (0, 0)),
            pl.BlockSpec((1, HID), lambda n: (0, 0)),
        ],
        out_specs=[
            pl.BlockSpec((4, B, HID), lambda n: (n, 0, 0)),
            pl.BlockSpec((4, B, HID), lambda n: (n, 0, 0)),
        ],
        out_shape=[out_sds, out_sds],
        compiler_params=pltpu.CompilerParams(
            dimension_semantics=("parallel",),
            vmem_limit_bytes=52 * 1024 * 1024,
        ),
        name="kv_pass",
    )(r, or_w2, or_b2_3d, k_w, k_b, v_w, v_b)


def _attn_kernel(q_ref, k_ref, v_ref, ow_ref, ob_ref, out_ref):
    q = q_ref[...]
    k = k_ref[...]
    v = v_ref[...]
    owb = ow_ref[...].astype(BF)
    ob = ob_ref[...]

    ii = jax.lax.broadcasted_iota(jnp.int32, (N * B, N * B), 0)
    jj = jax.lax.broadcasted_iota(jnp.int32, (N * B, N * B), 1)
    same_b = (ii & (B - 1)) == (jj & (B - 1))

    out = ob.astype(F32)
    for h in range(HEADS):
        sl = slice(h * HD, (h + 1) * HD)
        qh = q[:, sl]
        kh = k[:, sl]
        vh = v[:, sl]
        s = _dot_t(qh, kh) * (1.0 / SCALE)
        s = jnp.where(same_b, s, -1e30)
        s = s - jnp.max(s, axis=-1, keepdims=True)
        e = jnp.exp(s)
        p = e / jnp.sum(e, axis=-1, keepdims=True)
        ctx_h = _dot(p.astype(BF), vh)
        out = out + _dot(ctx_h, owb[sl, :])
    out_ref[...] = out.astype(BF).reshape(N, B, HID)


def _attn_pass(qfeat, k2d, v2d, o_w, o_b):
    return pl.pallas_call(
        _attn_kernel,
        out_shape=jax.ShapeDtypeStruct((N, B, HID), BF),
        name="attn_pass",
    )(qfeat, k2d, v2d, o_w, o_b)


def _head_kernel(ao_ref, w1_ref, atb1_ref, atg_ref, atbe_ref, atm_ref, atv_ref,
                 w2_ref, atb2_ref, objsum_ref, scores_ref, acc_ref):
    j = pl.program_id(0)

    @pl.when(j == 0)
    def _():
        acc_ref[...] = jnp.zeros_like(acc_ref)

    acc_ref[...] += (_dot(ao_ref[0], w1_ref[0]) + _dot(ao_ref[1], w1_ref[1])
                     + _dot(ao_ref[2], w1_ref[2]) + _dot(ao_ref[3], w1_ref[3]))

    @pl.when(j == N // 4 - 1)
    def _():
        x = acc_ref[...] + atb1_ref[...]
        s = jax.nn.relu((x - atm_ref[...]) * jax.lax.rsqrt(atv_ref[...] + 1e-5)
                        * atg_ref[...] + atbe_ref[...])
        sc = _dot(s, w2_ref[...]) + atb2_ref[...]
        scores_ref[...] = jnp.where(objsum_ref[...] == 0.0, jnp.float32(-1e-6), sc)


def _head_pass(attnout, at_w1_3d, at_b1, at_g, at_be, at_m, at_v,
               at_w2, at_b2, objsum2d):
    return pl.pallas_call(
        _head_kernel,
        grid=(N // 4,),
        in_specs=[
            pl.BlockSpec((4, B, HID), lambda n: (n, 0, 0)),
            pl.BlockSpec((4, HID, HID), lambda n: (n, 0, 0)),
            pl.BlockSpec((1, HID), lambda n: (0, 0)),
            pl.BlockSpec((1, HID), lambda n: (0, 0)),
            pl.BlockSpec((1, HID), lambda n: (0, 0)),
            pl.BlockSpec((1, HID), lambda n: (0, 0)),
            pl.BlockSpec((1, HID), lambda n: (0, 0)),
            pl.BlockSpec((HID, N), lambda n: (0, 0)),
            pl.BlockSpec((1, N), lambda n: (0, 0)),
            pl.BlockSpec((B, N), lambda n: (0, 0)),
        ],
        out_specs=pl.BlockSpec((B, N), lambda n: (0, 0)),
        out_shape=jax.ShapeDtypeStruct((B, N), F32),
        scratch_shapes=[pltpu.VMEM((B, HID), F32)],
        compiler_params=pltpu.CompilerParams(
            dimension_semantics=("arbitrary",),
            vmem_limit_bytes=52 * 1024 * 1024,
        ),
        name="head_pass",
    )(attnout, at_w1_3d, at_b1, at_g, at_be, at_m, at_v, at_w2, at_b2, objsum2d)


def kernel(scene_mask, target_mask, object_masks, bboxes,
           bb_w1, bb_b1, bb_fw, bb_fb,
           or_w1, or_b1, or_g, or_be, or_m, or_v, or_w2, or_b2,
           at_w1, at_b1, at_g, at_be, at_m, at_v, at_w2, at_b2,
           q_w, q_b, k_w, k_b, v_w, v_b, o_w, o_b):
    obj = object_masks.reshape(B, N, H, W)
    tgt = target_mask.reshape(B, H, W)
    bb_t = bboxes.transpose(0, 2, 1)

    objsum, overlap, iou = _mask_pass(obj, tgt, bb_t)
    objsum2d = objsum.reshape(B, N)

    row = lambda x: x.reshape(1, -1)
    objsum_col = objsum2d.T.reshape(N * B, 1)
    qfeat, r = _front_pass(
        objsum_col, bb_w1, row(bb_b1), bb_fw, row(bb_fb),
        overlap.reshape(B, N), iou.reshape(B, N),
        or_w1[0::2], or_w1[1::2], row(or_b1), row(or_g), row(or_be),
        row(or_m), row(or_v), q_w, row(q_b))

    k, v = _kv_pass(r, or_w2, or_b2.reshape(N, 1, HID),
                    k_w, row(k_b), v_w, row(v_b))

    attnout = _attn_pass(qfeat, k.reshape(N * B, HID), v.reshape(N * B, HID),
                         o_w, row(o_b))

    return _head_pass(attnout, at_w1.reshape(N, HID, HID),
                      row(at_b1), row(at_g), row(at_be), row(at_m), row(at_v),
                      at_w2, row(at_b2), objsum2d)


# speedup vs baseline: 1.0984x; 1.0984x over previous
import math

import jax
import jax.numpy as jnp
from jax.experimental import pallas as pl
from jax.experimental.pallas import tpu as pltpu

B, N, H, W = 16, 20, 224, 224
HID = 1024
HEADS = 4
HD = HID // HEADS
SCALE = float(math.sqrt(float(HD)))
INV_HW = 1.0 / float(H * W)
BF = jnp.bfloat16
F32 = jnp.float32


def _dot(a, b):
    return jnp.dot(a.astype(BF) if a.dtype != BF else a,
                   b.astype(BF) if b.dtype != BF else b,
                   preferred_element_type=F32)


def _dot_t(a, b):
    return jax.lax.dot_general(a, b, (((1,), (1,)), ((), ())),
                               preferred_element_type=F32)


def _mask_body(o, t, bb, objsum_ref, overlap_ref, iou_ref, i):
    objsum = jnp.sum(o, axis=(1, 2))
    overlap = jnp.sum(o * t[None, :, :], axis=(1, 2))

    x1 = jnp.floor(bb[0])
    y1 = jnp.floor(bb[1])
    x2 = jnp.floor(bb[2])
    y2 = jnp.floor(bb[3])

    hi = jax.lax.broadcasted_iota(jnp.int32, (H, N), 0).astype(F32)
    rowm = jnp.where((hi >= y1[None, :]) & (hi < y2[None, :]), 1.0, 0.0)
    wi = jax.lax.broadcasted_iota(jnp.int32, (W, N), 0).astype(F32)
    colm = jnp.where((wi >= x1[None, :]) & (wi < x2[None, :]), 1.0, 0.0)

    tmp = jnp.dot(t, colm, preferred_element_type=F32)
    inter = jnp.sum(rowm * tmp, axis=0)
    box_area = jnp.sum(rowm, axis=0) * jnp.sum(colm, axis=0)
    t_area = jnp.sum(t)
    iou = inter / (box_area + t_area - inter + 1e-8)

    objsum_ref[i, 0] = objsum
    overlap_ref[i, 0] = overlap
    iou_ref[i, 0] = iou


def _mask_kernel(obj_ref, tgt_ref, bb_ref, objsum_ref, overlap_ref, iou_ref):
    for i in range(2):
        _mask_body(obj_ref[i], tgt_ref[i], bb_ref[i],
                   objsum_ref, overlap_ref, iou_ref, i)


def _mask_pass(obj, tgt, bb_t):
    out_sds = jax.ShapeDtypeStruct((B, 1, N), F32)
    return pl.pallas_call(
        _mask_kernel,
        grid=(B // 2,),
        in_specs=[
            pl.BlockSpec((2, N, H, W), lambda b: (b, 0, 0, 0)),
            pl.BlockSpec((2, H, W), lambda b: (b, 0, 0)),
            pl.BlockSpec((2, 4, N), lambda b: (b, 0, 0)),
        ],
        out_specs=[
            pl.BlockSpec((2, 1, N), lambda b: (b, 0, 0)),
            pl.BlockSpec((2, 1, N), lambda b: (b, 0, 0)),
            pl.BlockSpec((2, 1, N), lambda b: (b, 0, 0)),
        ],
        out_shape=[out_sds, out_sds, out_sds],
        compiler_params=pltpu.CompilerParams(
            dimension_semantics=("parallel",),
        ),
        name="mask_pass",
    )(obj, tgt, bb_t)


def _frontkv_kernel(objc_ref, bbw1_ref, bbb1_ref, bbfw_ref, bbfb_ref,
                    ov_ref, iou_ref, w1ov_ref, w1iou_ref, orb1_ref,
                    org_ref, orbe_ref, orm_ref, orv_ref,
                    qw_ref, qb_ref,
                    w2_ref, b2_ref, kw_ref, kb_ref, vw_ref, vb_ref,
                    qfeat_ref, k_ref, v_ref, r_scr):
    j = pl.program_id(0)

    @pl.when(j == 0)
    def _():
        f = (objc_ref[...] * INV_HW).astype(BF).astype(F32)
        w1b = bbw1_ref[...].astype(BF).astype(F32)
        w1s = jnp.sum(w1b, axis=0, keepdims=True)
        h = jax.nn.relu(f * w1s + bbb1_ref[...])
        of = _dot(h, bbfw_ref[...]) + bbfb_ref[...]
        q = _dot(of, qw_ref[...]) + qb_ref[...]
        qfeat_ref[...] = q.astype(BF)

        e = (_dot(ov_ref[...], w1ov_ref[...])
             + _dot(iou_ref[...], w1iou_ref[...])
             + orb1_ref[...])
        bn = ((e - orm_ref[...]) * jax.lax.rsqrt(orv_ref[...] + 1e-5)
              * org_ref[...] + orbe_ref[...])
        r_scr[...] = jax.nn.relu(bn).astype(BF)

    rb = r_scr[...]
    kwb = kw_ref[...].astype(BF)
    vwb = vw_ref[...].astype(BF)
    for i in range(2):
        rel = _dot(rb, w2_ref[:, i * HID:(i + 1) * HID]) + b2_ref[i]
        k_ref[i] = (_dot(rel, kwb) + kb_ref[...]).astype(BF)
        v_ref[i] = (_dot(rel, vwb) + vb_ref[...]).astype(BF)


def _frontkv_pass(objsum_col, bb_w1, bb_b1, bb_fw, bb_fb,
                  overlap, iou, w1_ov, w1_iou, or_b1, or_g, or_be, or_m, or_v,
                  q_w, q_b, or_w2, or_b2_3d, k_w, k_b, v_w, v_b):
    kv_sds = jax.ShapeDtypeStruct((N, B, HID), BF)
    full = lambda a: pl.BlockSpec(a.shape, lambda n: (0,) * a.ndim)
    return pl.pallas_call(
        _frontkv_kernel,
        grid=(N // 2,),
        in_specs=[
            full(objsum_col), full(bb_w1), full(bb_b1), full(bb_fw),
            full(bb_fb), full(overlap), full(iou), full(w1_ov), full(w1_iou),
            full(or_b1), full(or_g), full(or_be), full(or_m), full(or_v),
            full(q_w), full(q_b),
            pl.BlockSpec((HID, 2 * HID), lambda n: (0, n)),
            pl.BlockSpec((2, 1, HID), lambda n: (n, 0, 0)),
            full(k_w), full(k_b), full(v_w), full(v_b),
        ],
        out_specs=[
            pl.BlockSpec((N * B, HID), lambda n: (0, 0)),
            pl.BlockSpec((2, B, HID), lambda n: (n, 0, 0)),
            pl.BlockSpec((2, B, HID), lambda n: (n, 0, 0)),
        ],
        out_shape=[jax.ShapeDtypeStruct((N * B, HID), BF), kv_sds, kv_sds],
        scratch_shapes=[pltpu.VMEM((B, HID), BF)],
        compiler_params=pltpu.CompilerParams(
            dimension_semantics=("arbitrary",),
            vmem_limit_bytes=52 * 1024 * 1024,
        ),
        name="frontkv_pass",
    )(objsum_col, bb_w1, bb_b1, bb_fw, bb_fb,
      overlap, iou, w1_ov, w1_iou, or_b1, or_g, or_be, or_m, or_v,
      q_w, q_b, or_w2, or_b2_3d, k_w, k_b, v_w, v_b)


def _attnhead_kernel(q_ref, k_ref, v_ref, ow_ref, ob_ref,
                     w1_ref, atb1_ref, atg_ref, atbe_ref, atm_ref, atv_ref,
                     w2_ref, atb2_ref, objsum_ref,
                     scores_ref, ao_scr, acc_ref):
    j = pl.program_id(0)

    @pl.when(j == 0)
    def _():
        q = q_ref[...]
        k = k_ref[...]
        v = v_ref[...]
        owb = ow_ref[...].astype(BF)

        ii = jax.lax.broadcasted_iota(jnp.int32, (N * B, N * B), 0)
        jj = jax.lax.broadcasted_iota(jnp.int32, (N * B, N * B), 1)
        same_b = (ii & (B - 1)) == (jj & (B - 1))

        out = ob_ref[...].astype(F32)
        for h in range(HEADS):
            sl = slice(h * HD, (h + 1) * HD)
            s = _dot_t(q[:, sl], k[:, sl]) * (1.0 / SCALE)
            s = jnp.where(same_b, s, -1e30)
            s = s - jnp.max(s, axis=-1, keepdims=True)
            e = jnp.exp(s)
            p = e / jnp.sum(e, axis=-1, keepdims=True)
            ctx_h = _dot(p.astype(BF), v[:, sl])
            out = out + _dot(ctx_h, owb[sl, :])
        ao_scr[...] = out.astype(BF).reshape(N, B, HID)
        acc_ref[...] = jnp.zeros_like(acc_ref)

    acc_ref[...] += (_dot(ao_scr[2 * j], w1_ref[0])
                     + _dot(ao_scr[2 * j + 1], w1_ref[1]))

    @pl.when(j == N // 2 - 1)
    def _():
        x = acc_ref[...] + atb1_ref[...]
        s = jax.nn.relu((x - atm_ref[...]) * jax.lax.rsqrt(atv_ref[...] + 1e-5)
                        * atg_ref[...] + atbe_ref[...])
        sc = _dot(s, w2_ref[...]) + atb2_ref[...]
        scores_ref[...] = jnp.where(objsum_ref[...] == 0.0, jnp.float32(-1e-6), sc)


def _attnhead_pass(qfeat, k2d, v2d, o_w, o_b, at_w1_3d,
                   at_b1, at_g, at_be, at_m, at_v, at_w2, at_b2, objsum2d):
    full = lambda a: pl.BlockSpec(a.shape, lambda n: (0,) * a.ndim)
    return pl.pallas_call(
        _attnhead_kernel,
        grid=(N // 2,),
        in_specs=[
            full(qfeat), full(k2d), full(v2d), full(o_w), full(o_b),
            pl.BlockSpec((2, HID, HID), lambda n: (n, 0, 0)),
            full(at_b1), full(at_g), full(at_be), full(at_m), full(at_v),
            full(at_w2), full(at_b2), full(objsum2d),
        ],
        out_specs=pl.BlockSpec((B, N), lambda n: (0, 0)),
        out_shape=jax.ShapeDtypeStruct((B, N), F32),
        scratch_shapes=[pltpu.VMEM((N, B, HID), BF), pltpu.VMEM((B, HID), F32)],
        compiler_params=pltpu.CompilerParams(
            dimension_semantics=("arbitrary",),
            vmem_limit_bytes=52 * 1024 * 1024,
        ),
        name="attnhead_pass",
    )(qfeat, k2d, v2d, o_w, o_b, at_w1_3d,
      at_b1, at_g, at_be, at_m, at_v, at_w2, at_b2, objsum2d)


def kernel(scene_mask, target_mask, object_masks, bboxes,
           bb_w1, bb_b1, bb_fw, bb_fb,
           or_w1, or_b1, or_g, or_be, or_m, or_v, or_w2, or_b2,
           at_w1, at_b1, at_g, at_be, at_m, at_v, at_w2, at_b2,
           q_w, q_b, k_w, k_b, v_w, v_b, o_w, o_b):
    obj = object_masks.reshape(B, N, H, W)
    tgt = target_mask.reshape(B, H, W)
    bb_t = bboxes.transpose(0, 2, 1)

    objsum, overlap, iou = _mask_pass(obj, tgt, bb_t)
    objsum2d = objsum.reshape(B, N)

    row = lambda x: x.reshape(1, -1)
    objsum_col = objsum2d.T.reshape(N * B, 1)
    qfeat, k, v = _frontkv_pass(
        objsum_col, bb_w1, row(bb_b1), bb_fw, row(bb_fb),
        overlap.reshape(B, N), iou.reshape(B, N),
        or_w1[0::2], or_w1[1::2], row(or_b1), row(or_g), row(or_be),
        row(or_m), row(or_v), q_w, row(q_b),
        or_w2, or_b2.reshape(N, 1, HID), k_w, row(k_b), v_w, row(v_b))

    return _attnhead_pass(qfeat, k.reshape(N * B, HID), v.reshape(N * B, HID),
                          o_w, row(o_b), at_w1.reshape(N, HID, HID),
                          row(at_b1), row(at_g), row(at_be), row(at_m),
                          row(at_v), at_w2, row(at_b2), objsum2d)
